# Initial kernel scaffold; baseline (speedup 1.0000x reference)
import jax
import jax.numpy as jnp
from jax.experimental import pallas as pl


def kernel(predictions, target, alpha):
    raise NotImplementedError("write your pallas kernel here")


# speedup vs baseline: 64.0493x; 64.0493x over previous
import functools

import jax
import jax.numpy as jnp
import numpy as np
from jax.experimental import pallas as pl
from jax.experimental.pallas import tpu as pltpu


def _roberts_sq(x):
    h, w = x.shape
    zrow = jnp.zeros((1, w), jnp.float32)
    zcol = jnp.zeros((h, 1), jnp.float32)
    below = jnp.concatenate([x[1:, :], zrow], axis=0)
    right = jnp.concatenate([x[:, 1:], zcol], axis=1)
    belowright = jnp.concatenate([below[:, 1:], zcol], axis=1)
    gx = x - belowright
    gy = right - below
    return gx * gx + gy * gy + jnp.float32(1e-12)


def _merge_body(ax, t_ref, p_ref, out_ref, tf_ref, pf_ref, acc_ref):
    i = pl.program_id(0)
    n = pl.num_programs(0)

    @pl.when(i == 0)
    def _init():
        tf_ref[...] = jnp.zeros_like(tf_ref)
        pf_ref[...] = jnp.zeros_like(pf_ref)
        acc_ref[0] = jnp.float32(0.0)

    tsq = _roberts_sq(t_ref[0])
    psq = _roberts_sq(p_ref[0])
    tbits = jax.lax.bitcast_convert_type(tsq, jnp.int32)
    pbits = jax.lax.bitcast_convert_type(psq, jnp.int32)

    def bisect(_, carry):
        tlo, thi, plo, phi = carry
        tmid = tlo + (thi - tlo) // 2
        pmid = plo + (phi - plo) // 2
        tc = jnp.sum((tbits >= tmid).astype(jnp.int32))
        pc = jnp.sum((pbits >= pmid).astype(jnp.int32))
        tge = tc >= ax
        pge = pc >= ax
        return (jnp.where(tge, tmid, tlo), jnp.where(tge, thi, tmid),
                jnp.where(pge, pmid, plo), jnp.where(pge, phi, pmid))

    hi0 = jnp.int32(0x40000000)
    tlo, _, plo, _ = jax.lax.fori_loop(
        0, 30, bisect, (jnp.int32(0), hi0, jnp.int32(0), hi0))

    tf = jnp.where(tbits >= tlo, jnp.sqrt(tsq), tf_ref[...])
    pf = jnp.where(pbits >= plo, jnp.sqrt(psq), pf_ref[...])
    tf_ref[...] = tf
    pf_ref[...] = pf
    e = jnp.abs((tf - pf) / (tf + pf + jnp.float32(1e-5)))
    acc_ref[0] += jnp.sum(e)

    @pl.when(i == n - 1)
    def _fin():
        out_ref[0] = acc_ref[0]


def kernel(predictions, target, alpha):
    b, c, h, w = predictions.shape
    n = b * c
    hw = h * w
    ax = int(np.floor(0.1 * hw))
    t_seq = target.reshape(n, h, w)
    p_seq = predictions.reshape(n, h, w)

    body = functools.partial(_merge_body, ax)
    total = pl.pallas_call(
        body,
        grid=(n,),
        in_specs=[
            pl.BlockSpec((1, h, w), lambda i: (i, 0, 0)),
            pl.BlockSpec((1, h, w), lambda i: (i, 0, 0)),
        ],
        out_specs=pl.BlockSpec(memory_space=pltpu.SMEM),
        out_shape=jax.ShapeDtypeStruct((1,), jnp.float32),
        scratch_shapes=[
            pltpu.VMEM((h, w), jnp.float32),
            pltpu.VMEM((h, w), jnp.float32),
            pltpu.SMEM((1,), jnp.float32),
        ],
    )(t_seq, p_seq)
    return alpha * total[0] / jnp.float32(n * hw)
